# baseline (speedup 1.0000x reference)
import functools

import jax
import jax.numpy as jnp
from jax import lax
from jax.experimental import pallas as pl
from jax.experimental.pallas import tpu as pltpu
from jax.experimental.pallas import tpu_sc as plsc

_NC = 2
_NS = 16
_NW = _NC * _NS

_K = 7936
_NBUF = 2


def _ceil_log2(n):
    b = 0
    while (1 << b) < n:
        b += 1
    return b


def _pack_body(*refs):
    o_ref = refs[-1]
    acc = refs[0][...]
    for r in range(1, 16):
        acc = acc | lax.shift_left(refs[r][...], 2 * r)
    o_ref[...] = acc


def _pack_types(at, tail, full_rows, cols):
    bc = min(cols, 8192)
    assert cols % bc == 0
    grid = cols // bc
    specs = []
    ops = []
    for r in range(16):
        if r < full_rows:
            specs.append(pl.BlockSpec(
                (bc,), functools.partial(lambda r, i: r * grid + i, r)))
            ops.append(at)
        else:
            specs.append(pl.BlockSpec(
                (bc,),
                functools.partial(lambda r, i: r * grid + i, r - full_rows)))
            ops.append(tail)
    packed = pl.pallas_call(
        _pack_body,
        grid=(grid,),
        in_specs=specs,
        out_specs=pl.BlockSpec((bc,), lambda i: (i,)),
        out_shape=jax.ShapeDtypeStruct((cols,), jnp.int32),
    )(*ops)
    return packed


def _make_sc_kernel(cols, shift_bits, n_bonds, k):
    chunks = -(-n_bonds // (_NW * k))
    mesh = plsc.VectorSubcoreMesh(core_axis_name="c", subcore_axis_name="s")
    cmask = jnp.int32(cols - 1)
    last_base = jnp.int32(n_bonds - k)

    @functools.partial(
        pl.kernel,
        out_type=jax.ShapeDtypeStruct((n_bonds,), jnp.float32),
        mesh=mesh,
        compiler_params=pltpu.CompilerParams(needs_layout_passes=False),
        scratch_types=(
            [pltpu.VMEM((cols,), jnp.int32)]
            + [pltpu.VMEM((k,), jnp.int32) for _ in range(2 * _NBUF)]
            + [pltpu.VMEM((k,), jnp.float32) for _ in range(2 * _NBUF)]
            + [pltpu.VMEM((16,), jnp.float32) for _ in range(2)]
            + [pltpu.SemaphoreType.DMA for _ in range(2 * _NBUF)]
        ),
    )
    def sc_kernel(packed_hbm, bi_hbm, bj_hbm, dist_hbm, kh_hbm, r0_hbm,
                  out_hbm, packed_v, *rest):
        bis = rest[0:_NBUF]
        bjs = rest[_NBUF:2 * _NBUF]
        bds = rest[2 * _NBUF:3 * _NBUF]
        ovs = rest[3 * _NBUF:4 * _NBUF]
        kh_v, r0_v = rest[4 * _NBUF:4 * _NBUF + 2]
        sins = rest[4 * _NBUF + 2:5 * _NBUF + 2]
        souts = rest[5 * _NBUF + 2:6 * _NBUF + 2]
        cid = lax.axis_index("c")
        sid = lax.axis_index("s")
        wid = sid * _NC + cid
        ins = tuple(
            (bis[p], bjs[p], bds[p], sins[p]) for p in range(_NBUF))
        outs = tuple((ovs[p], souts[p]) for p in range(_NBUF))

        def base_of(ci):
            g = wid * chunks + ci
            return pl.multiple_of(jnp.minimum(g * k, last_base), 8)

        def start_in(ci):
            b = base_of(ci)
            bi_v, bj_v, bd_v, sem = ins[ci % _NBUF]
            return (
                pltpu.async_copy(bi_hbm.at[pl.ds(b, k)], bi_v, sem),
                pltpu.async_copy(bj_hbm.at[pl.ds(b, k)], bj_v, sem),
                pltpu.async_copy(dist_hbm.at[pl.ds(b, k)], bd_v, sem),
            )

        def decode(b):
            w = plsc.load_gather(packed_v, [b & cmask])
            sh = (b >> shift_bits) << 1
            return (w >> sh) & jnp.int32(3)

        pending = {}
        for ci in range(min(_NBUF, chunks)):
            pending[ci] = start_in(ci)
        pltpu.sync_copy(packed_hbm, packed_v)
        pltpu.sync_copy(kh_hbm, kh_v)
        pltpu.sync_copy(r0_hbm, r0_v)

        h_out = [None] * _NBUF
        for ci in range(chunks):
            for h in pending.pop(ci):
                h.wait()
            if h_out[ci % _NBUF] is not None:
                h_out[ci % _NBUF].wait()
            bi_v, bj_v, bd_v, _ = ins[ci % _NBUF]
            out_v, sem_o = outs[ci % _NBUF]

            @plsc.parallel_loop(0, k, step=16, unroll=16)
            def vec_body(off):
                bi = bi_v[pl.ds(off, 16)]
                bj = bj_v[pl.ds(off, 16)]
                d = bd_v[pl.ds(off, 16)]
                ti = decode(bi)
                tj = decode(bj)
                comb = (ti << 2) | tj
                kh = plsc.load_gather(kh_v, [comb])
                r0 = plsc.load_gather(r0_v, [comb])
                dd = d - r0
                out_v[pl.ds(off, 16)] = kh * dd * dd

            h_out[ci % _NBUF] = pltpu.async_copy(
                out_v, out_hbm.at[pl.ds(base_of(ci), k)], sem_o)
            if ci + _NBUF < chunks:
                pending[ci + _NBUF] = start_in(ci + _NBUF)
        for h in h_out:
            if h is not None:
                h.wait()

    return sc_kernel


def kernel(atoms_type, bonds_i, bonds_j, bonds_dist, k_table, r0_table):
    n_atoms = atoms_type.shape[0]
    n_bonds = bonds_i.shape[0]

    cap_bits = max(_ceil_log2(n_atoms), 5)
    cap = 1 << cap_bits
    cols = cap // 16
    shift_bits = cap_bits - 4

    at = atoms_type.astype(jnp.int32)
    full_rows = min(n_atoms // cols, 16)
    if full_rows < 16:
        tail = jnp.pad(at[full_rows * cols:],
                       (0, (16 - full_rows) * cols - (n_atoms - full_rows * cols)))
    else:
        tail = at[:0]
    packed = _pack_types(at, tail, full_rows, cols)

    k = _K if n_bonds >= _K else ((n_bonds + 15) // 16) * 16
    rounded = ((n_bonds + 7) // 8) * 8
    pad = max(rounded, k) - n_bonds
    bi = bonds_i.astype(jnp.int32)
    bj = bonds_j.astype(jnp.int32)
    bd = bonds_dist
    if pad:
        bi = jnp.pad(bi, (0, pad))
        bj = jnp.pad(bj, (0, pad))
        bd = jnp.pad(bd, (0, pad))

    kh16 = (0.5 * k_table).astype(jnp.float32).reshape(16)
    r016 = r0_table.astype(jnp.float32).reshape(16)

    sc = _make_sc_kernel(cols, shift_bits, n_bonds + pad, k)
    out = sc(packed, bi, bj, bd, kh16, r016)
    return out[:n_bonds] if pad else out


# speedup vs baseline: 1.0524x; 1.0524x over previous
import functools

import jax
import jax.numpy as jnp
from jax import lax
from jax.experimental import pallas as pl
from jax.experimental.pallas import tpu as pltpu
from jax.experimental.pallas import tpu_sc as plsc

_NC = 2
_NS = 16
_NW = _NC * _NS

_K = 7936
_NBUF = 2


def _ceil_log2(n):
    b = 0
    while (1 << b) < n:
        b += 1
    return b


def _pack_body(*refs):
    o_ref = refs[-1]
    acc = refs[0][...]
    for r in range(1, 16):
        acc = acc | lax.shift_left(refs[r][...], 2 * r)
    o_ref[...] = acc


def _pack_types(at, tail, full_rows, cols):
    bc = min(cols, 16384)
    assert cols % bc == 0
    grid = cols // bc
    specs = []
    ops = []
    for r in range(16):
        if r < full_rows:
            specs.append(pl.BlockSpec(
                (bc,), functools.partial(lambda r, i: r * grid + i, r)))
            ops.append(at)
        else:
            specs.append(pl.BlockSpec(
                (bc,),
                functools.partial(lambda r, i: r * grid + i, r - full_rows)))
            ops.append(tail)
    packed = pl.pallas_call(
        _pack_body,
        grid=(grid,),
        in_specs=specs,
        out_specs=pl.BlockSpec((bc,), lambda i: (i,)),
        out_shape=jax.ShapeDtypeStruct((cols,), jnp.int32),
    )(*ops)
    return packed


def _make_sc_kernel(cols, shift_bits, n_bonds, k):
    chunks = -(-n_bonds // (_NW * k))
    mesh = plsc.VectorSubcoreMesh(core_axis_name="c", subcore_axis_name="s")
    cmask = jnp.int32(cols - 1)
    last_base = jnp.int32(n_bonds - k)

    @functools.partial(
        pl.kernel,
        out_type=jax.ShapeDtypeStruct((n_bonds,), jnp.float32),
        mesh=mesh,
        compiler_params=pltpu.CompilerParams(needs_layout_passes=False),
        scratch_types=(
            [pltpu.VMEM((cols,), jnp.int32)]
            + [pltpu.VMEM((k,), jnp.int32) for _ in range(2 * _NBUF)]
            + [pltpu.VMEM((k,), jnp.float32) for _ in range(2 * _NBUF)]
            + [pltpu.VMEM((16,), jnp.float32) for _ in range(2)]
            + [pltpu.SemaphoreType.DMA for _ in range(2 * _NBUF)]
        ),
    )
    def sc_kernel(packed_hbm, bi_hbm, bj_hbm, dist_hbm, kh_hbm, r0_hbm,
                  out_hbm, packed_v, *rest):
        bis = rest[0:_NBUF]
        bjs = rest[_NBUF:2 * _NBUF]
        bds = rest[2 * _NBUF:3 * _NBUF]
        ovs = rest[3 * _NBUF:4 * _NBUF]
        kh_v, r0_v = rest[4 * _NBUF:4 * _NBUF + 2]
        sins = rest[4 * _NBUF + 2:5 * _NBUF + 2]
        souts = rest[5 * _NBUF + 2:6 * _NBUF + 2]
        cid = lax.axis_index("c")
        sid = lax.axis_index("s")
        wid = sid * _NC + cid
        ins = tuple(
            (bis[p], bjs[p], bds[p], sins[p]) for p in range(_NBUF))
        outs = tuple((ovs[p], souts[p]) for p in range(_NBUF))

        def base_of(ci):
            g = wid * chunks + ci
            return pl.multiple_of(jnp.minimum(g * k, last_base), 8)

        def start_in(ci):
            b = base_of(ci)
            bi_v, bj_v, bd_v, sem = ins[ci % _NBUF]
            return (
                pltpu.async_copy(bi_hbm.at[pl.ds(b, k)], bi_v, sem),
                pltpu.async_copy(bj_hbm.at[pl.ds(b, k)], bj_v, sem),
                pltpu.async_copy(dist_hbm.at[pl.ds(b, k)], bd_v, sem),
            )

        def decode(b):
            w = plsc.load_gather(packed_v, [b & cmask])
            sh = (b >> shift_bits) << 1
            return (w >> sh) & jnp.int32(3)

        pending = {}
        for ci in range(min(_NBUF, chunks)):
            pending[ci] = start_in(ci)
        pltpu.sync_copy(packed_hbm, packed_v)
        pltpu.sync_copy(kh_hbm, kh_v)
        pltpu.sync_copy(r0_hbm, r0_v)

        h_out = [None] * _NBUF
        for ci in range(chunks):
            for h in pending.pop(ci):
                h.wait()
            if h_out[ci % _NBUF] is not None:
                h_out[ci % _NBUF].wait()
            bi_v, bj_v, bd_v, _ = ins[ci % _NBUF]
            out_v, sem_o = outs[ci % _NBUF]

            @plsc.parallel_loop(0, k, step=16, unroll=8)
            def vec_body(off):
                bi = bi_v[pl.ds(off, 16)]
                bj = bj_v[pl.ds(off, 16)]
                d = bd_v[pl.ds(off, 16)]
                ti = decode(bi)
                tj = decode(bj)
                comb = (ti << 2) | tj
                kh = plsc.load_gather(kh_v, [comb])
                r0 = plsc.load_gather(r0_v, [comb])
                dd = d - r0
                out_v[pl.ds(off, 16)] = kh * dd * dd

            h_out[ci % _NBUF] = pltpu.async_copy(
                out_v, out_hbm.at[pl.ds(base_of(ci), k)], sem_o)
            if ci + _NBUF < chunks:
                pending[ci + _NBUF] = start_in(ci + _NBUF)
        for h in h_out:
            if h is not None:
                h.wait()

    return sc_kernel


def kernel(atoms_type, bonds_i, bonds_j, bonds_dist, k_table, r0_table):
    n_atoms = atoms_type.shape[0]
    n_bonds = bonds_i.shape[0]

    cap_bits = max(_ceil_log2(n_atoms), 5)
    cap = 1 << cap_bits
    cols = cap // 16
    shift_bits = cap_bits - 4

    at = atoms_type.astype(jnp.int32)
    full_rows = min(n_atoms // cols, 16)
    if full_rows < 16:
        tail = jnp.pad(at[full_rows * cols:],
                       (0, (16 - full_rows) * cols - (n_atoms - full_rows * cols)))
    else:
        tail = at[:0]
    packed = _pack_types(at, tail, full_rows, cols)

    k = _K if n_bonds >= _K else ((n_bonds + 15) // 16) * 16
    rounded = ((n_bonds + 7) // 8) * 8
    pad = max(rounded, k) - n_bonds
    bi = bonds_i.astype(jnp.int32)
    bj = bonds_j.astype(jnp.int32)
    bd = bonds_dist
    if pad:
        bi = jnp.pad(bi, (0, pad))
        bj = jnp.pad(bj, (0, pad))
        bd = jnp.pad(bd, (0, pad))

    kh16 = (0.5 * k_table).astype(jnp.float32).reshape(16)
    r016 = r0_table.astype(jnp.float32).reshape(16)

    sc = _make_sc_kernel(cols, shift_bits, n_bonds + pad, k)
    out = sc(packed, bi, bj, bd, kh16, r016)
    return out[:n_bonds] if pad else out


# speedup vs baseline: 1.0748x; 1.0213x over previous
import functools

import jax
import jax.numpy as jnp
from jax import lax
from jax.experimental import pallas as pl
from jax.experimental.pallas import tpu as pltpu
from jax.experimental.pallas import tpu_sc as plsc

_NC = 2
_NS = 16
_NW = _NC * _NS

_K = 7936
_NBUF = 2


def _ceil_log2(n):
    b = 0
    while (1 << b) < n:
        b += 1
    return b


def _pack_body(*refs):
    o_ref = refs[-1]
    acc = refs[0][...]
    for r in range(1, 16):
        acc = acc | lax.shift_left(refs[r][...], 2 * r)
    o_ref[...] = acc


def _pack_types(at, tail, full_rows, cols):
    bc = min(cols, 32768)
    assert cols % bc == 0
    grid = cols // bc
    specs = []
    ops = []
    for r in range(16):
        if r < full_rows:
            specs.append(pl.BlockSpec(
                (bc,), functools.partial(lambda r, i: r * grid + i, r)))
            ops.append(at)
        else:
            specs.append(pl.BlockSpec(
                (bc,),
                functools.partial(lambda r, i: r * grid + i, r - full_rows)))
            ops.append(tail)
    packed = pl.pallas_call(
        _pack_body,
        grid=(grid,),
        in_specs=specs,
        out_specs=pl.BlockSpec((bc,), lambda i: (i,)),
        out_shape=jax.ShapeDtypeStruct((cols,), jnp.int32),
    )(*ops)
    return packed


def _make_sc_kernel(cols, shift_bits, n_bonds, k):
    chunks = -(-n_bonds // (_NW * k))
    mesh = plsc.VectorSubcoreMesh(core_axis_name="c", subcore_axis_name="s")
    cmask = jnp.int32(cols - 1)
    last_base = jnp.int32(n_bonds - k)

    @functools.partial(
        pl.kernel,
        out_type=jax.ShapeDtypeStruct((n_bonds,), jnp.float32),
        mesh=mesh,
        compiler_params=pltpu.CompilerParams(needs_layout_passes=False),
        scratch_types=(
            [pltpu.VMEM((cols,), jnp.int32)]
            + [pltpu.VMEM((k,), jnp.int32) for _ in range(2 * _NBUF)]
            + [pltpu.VMEM((k,), jnp.float32) for _ in range(2 * _NBUF)]
            + [pltpu.VMEM((16,), jnp.float32) for _ in range(2)]
            + [pltpu.SemaphoreType.DMA for _ in range(2 * _NBUF)]
        ),
    )
    def sc_kernel(packed_hbm, bi_hbm, bj_hbm, dist_hbm, kh_hbm, r0_hbm,
                  out_hbm, packed_v, *rest):
        bis = rest[0:_NBUF]
        bjs = rest[_NBUF:2 * _NBUF]
        bds = rest[2 * _NBUF:3 * _NBUF]
        ovs = rest[3 * _NBUF:4 * _NBUF]
        kh_v, r0_v = rest[4 * _NBUF:4 * _NBUF + 2]
        sins = rest[4 * _NBUF + 2:5 * _NBUF + 2]
        souts = rest[5 * _NBUF + 2:6 * _NBUF + 2]
        cid = lax.axis_index("c")
        sid = lax.axis_index("s")
        wid = sid * _NC + cid
        ins = tuple(
            (bis[p], bjs[p], bds[p], sins[p]) for p in range(_NBUF))
        outs = tuple((ovs[p], souts[p]) for p in range(_NBUF))

        def base_of(ci):
            g = wid * chunks + ci
            return pl.multiple_of(jnp.minimum(g * k, last_base), 8)

        def start_in(ci):
            b = base_of(ci)
            bi_v, bj_v, bd_v, sem = ins[ci % _NBUF]
            return (
                pltpu.async_copy(bi_hbm.at[pl.ds(b, k)], bi_v, sem),
                pltpu.async_copy(bj_hbm.at[pl.ds(b, k)], bj_v, sem),
                pltpu.async_copy(dist_hbm.at[pl.ds(b, k)], bd_v, sem),
            )

        def decode(b):
            w = plsc.load_gather(packed_v, [b & cmask])
            sh = (b >> shift_bits) << 1
            return (w >> sh) & jnp.int32(3)

        pending = {}
        for ci in range(min(_NBUF, chunks)):
            pending[ci] = start_in(ci)
        pltpu.sync_copy(packed_hbm, packed_v)
        pltpu.sync_copy(kh_hbm, kh_v)
        pltpu.sync_copy(r0_hbm, r0_v)

        h_out = [None] * _NBUF
        for ci in range(chunks):
            for h in pending.pop(ci):
                h.wait()
            if h_out[ci % _NBUF] is not None:
                h_out[ci % _NBUF].wait()
            bi_v, bj_v, bd_v, _ = ins[ci % _NBUF]
            out_v, sem_o = outs[ci % _NBUF]

            @plsc.parallel_loop(0, k, step=16, unroll=8)
            def vec_body(off):
                bi = bi_v[pl.ds(off, 16)]
                bj = bj_v[pl.ds(off, 16)]
                d = bd_v[pl.ds(off, 16)]
                ti = decode(bi)
                tj = decode(bj)
                comb = (ti << 2) | tj
                kh = plsc.load_gather(kh_v, [comb])
                r0 = plsc.load_gather(r0_v, [comb])
                dd = d - r0
                out_v[pl.ds(off, 16)] = kh * dd * dd

            h_out[ci % _NBUF] = pltpu.async_copy(
                out_v, out_hbm.at[pl.ds(base_of(ci), k)], sem_o)
            if ci + _NBUF < chunks:
                pending[ci + _NBUF] = start_in(ci + _NBUF)
        for h in h_out:
            if h is not None:
                h.wait()

    return sc_kernel


def kernel(atoms_type, bonds_i, bonds_j, bonds_dist, k_table, r0_table):
    n_atoms = atoms_type.shape[0]
    n_bonds = bonds_i.shape[0]

    cap_bits = max(_ceil_log2(n_atoms), 5)
    cap = 1 << cap_bits
    cols = cap // 16
    shift_bits = cap_bits - 4

    at = atoms_type.astype(jnp.int32)
    full_rows = min(n_atoms // cols, 16)
    if full_rows < 16:
        tail = jnp.pad(at[full_rows * cols:],
                       (0, (16 - full_rows) * cols - (n_atoms - full_rows * cols)))
    else:
        tail = at[:0]
    packed = _pack_types(at, tail, full_rows, cols)

    k = _K if n_bonds >= _K else ((n_bonds + 15) // 16) * 16
    rounded = ((n_bonds + 7) // 8) * 8
    pad = max(rounded, k) - n_bonds
    bi = bonds_i.astype(jnp.int32)
    bj = bonds_j.astype(jnp.int32)
    bd = bonds_dist
    if pad:
        bi = jnp.pad(bi, (0, pad))
        bj = jnp.pad(bj, (0, pad))
        bd = jnp.pad(bd, (0, pad))

    kh16 = (0.5 * k_table).astype(jnp.float32).reshape(16)
    r016 = r0_table.astype(jnp.float32).reshape(16)

    sc = _make_sc_kernel(cols, shift_bits, n_bonds + pad, k)
    out = sc(packed, bi, bj, bd, kh16, r016)
    return out[:n_bonds] if pad else out
